# baseline (speedup 1.0000x reference)
import functools

import jax
import jax.numpy as jnp
from jax import lax
from jax.experimental import pallas as pl
from jax.experimental.pallas import tpu as pltpu
from jax.experimental.pallas import tpu_sc as plsc

_NC, _NS = 2, 16
_NW = _NC * _NS
_N_U, _N_I, _D = 1_000_000, 100_000, 64

_CW = 640
_T_I = _N_I // _CW
_REM = _N_I - _T_I * _CW
_G_I = -(-_T_I // _NW)

_mesh = plsc.VectorSubcoreMesh(core_axis_name="c", subcore_axis_name="s")


@functools.partial(
    pl.kernel,
    out_type=jax.ShapeDtypeStruct((_D, _N_I), jnp.float32),
    mesh=_mesh,
    scratch_types=[
        pltpu.VMEM((_D, _CW), jnp.float32),
        pltpu.VMEM((_D, _CW), jnp.float32),
        pltpu.SemaphoreType.DMA((2,)),
        pltpu.SemaphoreType.DMA((2,)),
    ],
)
def _sc_item_copy(i_hbm, out_i, buf0, buf1, in_sems, out_sems):
    wid = lax.axis_index("s") * _NC + lax.axis_index("c")
    bufs = (buf0, buf1)

    def valid(g):
        return g * _NW + wid < _T_I

    def gather(g, b):
        start = pl.multiple_of((g * _NW + wid) * _CW, 128)
        return pltpu.make_async_copy(
            i_hbm.at[:, pl.ds(start, _CW)], bufs[b], in_sems.at[b])

    def scatter(g, b):
        start = pl.multiple_of((g * _NW + wid) * _CW, 128)
        return pltpu.make_async_copy(
            bufs[b], out_i.at[:, pl.ds(start, _CW)], out_sems.at[b])

    def do(g, action):
        @pl.when(valid(g))
        def _():
            action()

    n = _G_I
    do(0, lambda: gather(0, 0).start())
    for g in range(n):
        b = g % 2
        do(g, lambda g=g, b=b: gather(g, b).wait())
        do(g, lambda g=g, b=b: scatter(g, b).start())
        if g + 1 < n:
            if g >= 1:
                do(g - 1, lambda g=g, b=b: scatter(g - 1, 1 - b).wait())
            do(g + 1, lambda g=g, b=b: gather(g + 1, 1 - b).start())
    if n >= 2:
        do(n - 2, lambda: scatter(n - 2, (n - 2) % 2).wait())
    do(n - 1, lambda: scatter(n - 1, (n - 1) % 2).wait())


_UB = 32768
_UG = -(-_N_U // _UB)


_NSTR = 4


def _tc_body(src, dst, sems):
    h = _D // _NSTR
    for k in range(_NSTR):
        pltpu.make_async_copy(
            src.at[pl.ds(k * h, h)], dst.at[pl.ds(k * h, h)], sems.at[k]).start()
    for k in range(_NSTR):
        pltpu.make_async_copy(
            src.at[pl.ds(k * h, h)], dst.at[pl.ds(k * h, h)], sems.at[k]).wait()


_tc_user_copy = pl.pallas_call(
    _tc_body,
    in_specs=[pl.BlockSpec(memory_space=pltpu.HBM)],
    out_specs=pl.BlockSpec(memory_space=pltpu.HBM),
    out_shape=jax.ShapeDtypeStruct((_D, _N_U), jnp.float32),
    scratch_shapes=[pltpu.SemaphoreType.DMA((_NSTR,))],
)


def kernel(emb_user, emb_item):
    u_t, i_t = emb_user.T, emb_item.T
    out_u = _tc_user_copy(u_t)
    out_i = _sc_item_copy(i_t)
    out_i = lax.dynamic_update_slice(out_i, i_t[:, _T_I * _CW:], (0, _T_I * _CW))
    return (out_u.T, out_i.T)


# speedup vs baseline: 40.5039x; 40.5039x over previous
import functools

import jax
import jax.numpy as jnp
from jax import lax
from jax.experimental import pallas as pl
from jax.experimental.pallas import tpu as pltpu
from jax.experimental.pallas import tpu_sc as plsc

_NC, _NS = 2, 16
_NW = _NC * _NS
_N_U, _N_I, _D = 1_000_000, 100_000, 64

_CW = 640
_T_I = _N_I // _CW
_REM = _N_I - _T_I * _CW
_G_I = -(-_T_I // _NW)

_mesh = plsc.VectorSubcoreMesh(core_axis_name="c", subcore_axis_name="s")


@functools.partial(
    pl.kernel,
    out_type=jax.ShapeDtypeStruct((_D, _N_I), jnp.float32),
    mesh=_mesh,
    scratch_types=[
        pltpu.VMEM((_D, _CW), jnp.float32),
        pltpu.VMEM((_D, _CW), jnp.float32),
        pltpu.SemaphoreType.DMA((2,)),
        pltpu.SemaphoreType.DMA((2,)),
    ],
)
def _sc_item_copy(i_hbm, out_i, buf0, buf1, in_sems, out_sems):
    wid = lax.axis_index("s") * _NC + lax.axis_index("c")
    bufs = (buf0, buf1)

    def valid(g):
        return g * _NW + wid < _T_I

    def gather(g, b):
        start = pl.multiple_of((g * _NW + wid) * _CW, 128)
        return pltpu.make_async_copy(
            i_hbm.at[:, pl.ds(start, _CW)], bufs[b], in_sems.at[b])

    def scatter(g, b):
        start = pl.multiple_of((g * _NW + wid) * _CW, 128)
        return pltpu.make_async_copy(
            bufs[b], out_i.at[:, pl.ds(start, _CW)], out_sems.at[b])

    def do(g, action):
        @pl.when(valid(g))
        def _():
            action()

    n = _G_I
    do(0, lambda: gather(0, 0).start())
    for g in range(n):
        b = g % 2
        do(g, lambda g=g, b=b: gather(g, b).wait())
        do(g, lambda g=g, b=b: scatter(g, b).start())
        if g + 1 < n:
            if g >= 1:
                do(g - 1, lambda g=g, b=b: scatter(g - 1, 1 - b).wait())
            do(g + 1, lambda g=g, b=b: gather(g + 1, 1 - b).start())
    if n >= 2:
        do(n - 2, lambda: scatter(n - 2, (n - 2) % 2).wait())
    do(n - 1, lambda: scatter(n - 1, (n - 1) % 2).wait())


_UB = 32768
_UG = -(-_N_U // _UB)


def _tc_body(src, dst):
    dst[...] = src[...]


_tc_user_copy = pl.pallas_call(
    _tc_body,
    grid=(_UG,),
    in_specs=[pl.BlockSpec((_D, _UB), lambda i: (0, i))],
    out_specs=pl.BlockSpec((_D, _UB), lambda i: (0, i)),
    out_shape=jax.ShapeDtypeStruct((_D, _N_U), jnp.float32),
    compiler_params=pltpu.CompilerParams(dimension_semantics=("parallel",)),
)


def kernel(emb_user, emb_item):
    u_t, i_t = emb_user.T, emb_item.T
    out_u = _tc_user_copy(u_t)
    out_i = _sc_item_copy(i_t)
    out_i = lax.dynamic_update_slice(out_i, i_t[:, _T_I * _CW:], (0, _T_I * _CW))
    return (out_u.T, out_i.T)
